# baseline (speedup 1.0000x reference)
import jax
import jax.numpy as jnp
from jax import lax
from jax.experimental import pallas as pl
from jax.experimental.pallas import tpu as pltpu
from jax.experimental.pallas import tpu_sc as plsc

B = 16384
F = 26
FD = 13
VOCAB = 100000

NC = 2
NS = 16
NW = NC * NS
BPW = B // NW
NIDX = BPW * F
CH = 128
NGC = NIDX // CH
BLK = 512


def _gather_body(idx_hbm, tab_hbm, out_hbm, idx_v, val_v, sem):
    wid = lax.axis_index("s") * NC + lax.axis_index("c")
    base = wid * NIDX
    pltpu.sync_copy(idx_hbm.at[pl.ds(base, NIDX)], idx_v)

    def fire(g, carry):
        sl = pl.ds(g * CH, CH)
        pltpu.async_copy(tab_hbm.at[idx_v.at[sl]], val_v.at[sl], sem)
        return carry

    lax.fori_loop(0, NGC, fire, 0)

    def drain(g, carry):
        sl = pl.ds(0, CH)
        pltpu.make_async_copy(tab_hbm.at[idx_v.at[sl]], val_v.at[sl],
                              sem).wait()
        return carry

    lax.fori_loop(0, NGC, drain, 0)
    pltpu.sync_copy(val_v, out_hbm.at[pl.ds(base, NIDX)])


def _finish_body(val_ref, mx_ref, dn_ref, dw_ref, b_ref, out_ref):
    sparse = jnp.sum(val_ref[...] * mx_ref[...], axis=1)
    dense = jnp.sum(dn_ref[...] * dw_ref[...], axis=1)
    out_ref[...] = sparse + dense + b_ref[0]


@jax.jit
def _run(idx1d, tab, mx, dn, dw2, br):
    mesh = plsc.VectorSubcoreMesh(core_axis_name="c", subcore_axis_name="s")
    vals = pl.kernel(
        _gather_body,
        out_type=jax.ShapeDtypeStruct((B * F,), jnp.float32),
        mesh=mesh,
        scratch_types=[
            pltpu.VMEM((NIDX,), jnp.int32),
            pltpu.VMEM((NIDX,), jnp.float32),
            pltpu.SemaphoreType.DMA,
        ],
    )(idx1d, tab)
    vals2d = vals.reshape(B, F)
    return pl.pallas_call(
        _finish_body,
        out_shape=jax.ShapeDtypeStruct((B,), jnp.float32),
        grid=(B // BLK,),
        in_specs=[
            pl.BlockSpec((BLK, F), lambda i: (i, 0)),
            pl.BlockSpec((BLK, F), lambda i: (i, 0)),
            pl.BlockSpec((BLK, FD), lambda i: (i, 0)),
            pl.BlockSpec((1, FD), lambda i: (0, 0)),
            pl.BlockSpec(memory_space=pltpu.SMEM),
        ],
        out_specs=pl.BlockSpec((BLK,), lambda i: (i,)),
    )(vals2d, mx, dn, dw2, br)


def kernel(sparse_idx, mx, dense_vals, sparse_table, dense_w, b):
    si = sparse_idx.astype(jnp.int32)
    idx1d = (si + (jnp.arange(F, dtype=jnp.int32) * VOCAB)[None, :]).reshape(-1)
    tab = sparse_table.reshape(-1)
    dw2 = dense_w[None, :]
    br = b.astype(jnp.float32)
    return _run(idx1d, tab, mx, dense_vals, dw2, br)


# speedup vs baseline: 1.7143x; 1.7143x over previous
import jax
import jax.numpy as jnp
from jax import lax
from jax.experimental import pallas as pl
from jax.experimental.pallas import tpu as pltpu
from jax.experimental.pallas import tpu_sc as plsc

B = 16384
F = 26
FD = 13
VOCAB = 100000

NC = 2
NS = 16
NW = NC * NS
BPW = B // NW
L = 16
NIDX = BPW * F
CH = 128
NGC = NIDX // CH
JS = BPW // L


def _sc_body(idx_hbm, mxt_hbm, dnt_hbm, tab_hbm, dwr_hbm, br_hbm, out_hbm,
             idx_v, val_v, mx_v, dn_v, dwr_v, br_v, acc_v, sem):
    wid = lax.axis_index("s") * NC + lax.axis_index("c")
    row0 = wid * BPW

    pltpu.sync_copy(idx_hbm.at[pl.ds(wid * NIDX, NIDX)], idx_v)

    def fire(g, carry):
        sl = pl.ds(g * CH, CH)
        pltpu.async_copy(tab_hbm.at[idx_v.at[sl]], val_v.at[sl], sem)
        return carry

    lax.fori_loop(0, NGC, fire, 0)

    pltpu.sync_copy(mxt_hbm.at[:, pl.ds(row0, BPW)], mx_v)
    pltpu.sync_copy(dnt_hbm.at[:, pl.ds(row0, BPW)], dn_v)
    pltpu.sync_copy(dwr_hbm, dwr_v)
    pltpu.sync_copy(br_hbm, br_v)

    def drain(g, carry):
        sl = pl.ds(0, CH)
        pltpu.make_async_copy(tab_hbm.at[idx_v.at[sl]], val_v.at[sl],
                              sem).wait()
        return carry

    lax.fori_loop(0, NGC, drain, 0)

    brv = br_v[...]

    def jstep(j, carry):
        col = j * L

        def facc(f, a):
            return a + val_v[pl.ds(f * BPW + col, L)] * mx_v[f, pl.ds(col, L)]

        acc = lax.fori_loop(0, F, facc, brv)

        def dacc(f, a):
            return a + dn_v[f, pl.ds(col, L)] * dwr_v[pl.ds(f * L, L)]

        acc = lax.fori_loop(0, FD, dacc, acc)
        acc_v[pl.ds(col, L)] = acc
        return carry

    lax.fori_loop(0, JS, jstep, 0)

    pltpu.sync_copy(acc_v, out_hbm.at[pl.ds(row0, BPW)])


@jax.jit
def _sc_call(idx_r, mxt, dnt, tab, dwr, br):
    mesh = plsc.VectorSubcoreMesh(core_axis_name="c", subcore_axis_name="s")
    return pl.kernel(
        _sc_body,
        out_type=jax.ShapeDtypeStruct((B,), jnp.float32),
        mesh=mesh,
        scratch_types=[
            pltpu.VMEM((NIDX,), jnp.int32),
            pltpu.VMEM((NIDX,), jnp.float32),
            pltpu.VMEM((F, BPW), jnp.float32),
            pltpu.VMEM((FD, BPW), jnp.float32),
            pltpu.VMEM((FD * L,), jnp.float32),
            pltpu.VMEM((L,), jnp.float32),
            pltpu.VMEM((BPW,), jnp.float32),
            pltpu.SemaphoreType.DMA,
        ],
    )(idx_r, mxt, dnt, tab, dwr, br)


def kernel(sparse_idx, mx, dense_vals, sparse_table, dense_w, b):
    si = sparse_idx.astype(jnp.int32)
    flat = si + (jnp.arange(F, dtype=jnp.int32) * VOCAB)[None, :]
    idx_r = flat.reshape(NW, BPW, F).transpose(0, 2, 1).reshape(-1)
    mxt = mx.T
    dnt = dense_vals.T
    tab = sparse_table.reshape(-1)
    dwr = jnp.broadcast_to(dense_w[:, None], (FD, L)).reshape(-1)
    br = jnp.broadcast_to(b, (L,)).astype(jnp.float32)
    return _sc_call(idx_r, mxt, dnt, tab, dwr, br)
